# baseline (speedup 1.0000x reference)
import functools

import jax
import jax.numpy as jnp
from jax import lax
from jax.experimental import pallas as pl
from jax.experimental.pallas import tpu as pltpu
from jax.experimental.pallas import tpu_sc as plsc

VOCAB = 100000
D = 128
B = 4
T_TEXT = 2048
T_ADD = 512
T_OUT = T_TEXT + T_ADD
NW = 32
WPB = NW // B
TXT_W = T_TEXT // WPB
ADD_W = T_ADD // WPB

_mesh = plsc.VectorSubcoreMesh(core_axis_name="c", subcore_axis_name="s")


@functools.partial(
    pl.kernel,
    out_type=[
        jax.ShapeDtypeStruct((B, T_OUT, D), jnp.float32),
        jax.ShapeDtypeStruct((B, T_OUT), jnp.int32),
    ],
    mesh=_mesh,
    scratch_types=[
        pltpu.VMEM((B, TXT_W), jnp.int32),
        pltpu.VMEM((B, 2 * ADD_W), jnp.int32),
        pltpu.VMEM((TXT_W, D), jnp.float32),
        pltpu.VMEM((ADD_W, D), jnp.float32),
        pltpu.SemaphoreType.DMA,
        pltpu.SemaphoreType.DMA,
        pltpu.SemaphoreType.DMA,
        pltpu.SemaphoreType.DMA,
    ],
)
def _gather_concat(ids_hbm, aids_hbm, am_hbm, addm_hbm, w_hbm,
                   out_emb, out_mask,
                   ids_v, aids_v, trows_v, arows_v,
                   g0sem, g1sem, osem, msem):
    wid = lax.axis_index("s") * 2 + lax.axis_index("c")
    b = wid // WPB
    j = wid % WPB
    toff = j * TXT_W
    aoff = T_TEXT + j * ADD_W

    pltpu.async_copy(ids_hbm.at[:, pl.ds(toff, TXT_W)], ids_v, g0sem)
    apair = (j // 2) * 2 * ADD_W
    acol = (j % 2) * ADD_W
    pltpu.async_copy(aids_hbm.at[:, pl.ds(apair, 2 * ADD_W)], aids_v, g1sem)

    is_mask_worker = wid == 0

    @pl.when(is_mask_worker)
    def _():
        pltpu.async_copy(am_hbm, out_mask.at[:, pl.ds(0, T_TEXT)], msem)
        pltpu.async_copy(addm_hbm, out_mask.at[:, pl.ds(T_TEXT, T_ADD)], msem)

    pltpu.make_async_copy(ids_hbm.at[:, pl.ds(0, TXT_W)], ids_v,
                          g0sem).wait()
    pltpu.make_async_copy(aids_hbm.at[:, pl.ds(0, 2 * ADD_W)], aids_v,
                          g1sem).wait()

    g_t0 = pltpu.async_copy(w_hbm.at[ids_v.at[b, pl.ds(0, 128)]],
                            trows_v.at[pl.ds(0, 128)], g0sem)
    g_t1 = pltpu.async_copy(w_hbm.at[ids_v.at[b, pl.ds(128, 128)]],
                            trows_v.at[pl.ds(128, 128)], g0sem)
    g_a = pltpu.async_copy(w_hbm.at[aids_v.at[b, pl.ds(acol, ADD_W)]],
                           arows_v, g1sem)


    g_t0.wait()
    g_t1.wait()
    w_t = pltpu.async_copy(trows_v, out_emb.at[b, pl.ds(toff, TXT_W)], osem)
    g_a.wait()
    w_a = pltpu.async_copy(arows_v, out_emb.at[b, pl.ds(aoff, ADD_W)], osem)
    w_t.wait()
    w_a.wait()

    @pl.when(is_mask_worker)
    def _():
        pltpu.make_async_copy(am_hbm, out_mask.at[:, pl.ds(0, T_TEXT)],
                              msem).wait()
        pltpu.make_async_copy(addm_hbm, out_mask.at[:, pl.ds(T_TEXT, T_ADD)],
                              msem).wait()


def kernel(input_ids, attention_mask, add_ids, add_mask, W):
    emb, mask = _gather_concat(input_ids, add_ids, attention_mask, add_mask,
                               W)
    return emb, mask


# speedup vs baseline: 1.0189x; 1.0189x over previous
import functools

import jax
import jax.numpy as jnp
from jax import lax
from jax.experimental import pallas as pl
from jax.experimental.pallas import tpu as pltpu
from jax.experimental.pallas import tpu_sc as plsc

VOCAB = 100000
D = 128
B = 4
T_TEXT = 2048
T_ADD = 512
T_OUT = T_TEXT + T_ADD
NW = 32
WPB = NW // B
TXT_W = T_TEXT // WPB
ADD_W = T_ADD // WPB

_mesh = plsc.VectorSubcoreMesh(core_axis_name="c", subcore_axis_name="s")


@functools.partial(
    pl.kernel,
    out_type=[
        jax.ShapeDtypeStruct((B, T_OUT, D), jnp.float32),
        jax.ShapeDtypeStruct((B, T_OUT), jnp.int32),
    ],
    mesh=_mesh,
    scratch_types=[
        pltpu.VMEM((B, TXT_W), jnp.int32),
        pltpu.VMEM((B, 2 * ADD_W), jnp.int32),
        pltpu.VMEM((TXT_W, D), jnp.float32),
        pltpu.VMEM((ADD_W, D), jnp.float32),
        pltpu.SemaphoreType.DMA,
        pltpu.SemaphoreType.DMA,
        pltpu.SemaphoreType.DMA,
        pltpu.SemaphoreType.DMA,
        pltpu.SemaphoreType.DMA,
    ],
)
def _gather_concat(ids_hbm, aids_hbm, am_hbm, addm_hbm, w_hbm,
                   out_emb, out_mask,
                   ids_v, aids_v, trows_v, arows_v,
                   g0sem, g1sem, g2sem, osem, msem):
    wid = lax.axis_index("s") * 2 + lax.axis_index("c")
    b = wid // WPB
    j = wid % WPB
    toff = j * TXT_W
    aoff = T_TEXT + j * ADD_W

    pltpu.async_copy(ids_hbm.at[:, pl.ds(toff, TXT_W)], ids_v, g0sem)
    apair = (j // 2) * 2 * ADD_W
    acol = (j % 2) * ADD_W
    pltpu.async_copy(aids_hbm.at[:, pl.ds(apair, 2 * ADD_W)], aids_v, g1sem)

    is_mask_worker = wid == 0

    @pl.when(is_mask_worker)
    def _():
        pltpu.async_copy(am_hbm, out_mask.at[:, pl.ds(0, T_TEXT)], msem)
        pltpu.async_copy(addm_hbm, out_mask.at[:, pl.ds(T_TEXT, T_ADD)], msem)

    pltpu.make_async_copy(ids_hbm.at[:, pl.ds(0, TXT_W)], ids_v,
                          g0sem).wait()
    pltpu.make_async_copy(aids_hbm.at[:, pl.ds(0, 2 * ADD_W)], aids_v,
                          g1sem).wait()

    chunks = [
        (aids_v.at[b, pl.ds(acol, ADD_W)], arows_v,
         out_emb.at[b, pl.ds(aoff, ADD_W)], g2sem),
        (ids_v.at[b, pl.ds(0, 128)], trows_v.at[pl.ds(0, 128)],
         out_emb.at[b, pl.ds(toff, 128)], g0sem),
        (ids_v.at[b, pl.ds(128, 128)], trows_v.at[pl.ds(128, 128)],
         out_emb.at[b, pl.ds(toff + 128, 128)], g1sem),
    ]
    gathers = [pltpu.async_copy(w_hbm.at[idx], buf, sem)
               for idx, buf, _, sem in chunks]


    outs = []
    for g, (_, buf, dst, _) in zip(gathers, chunks):
        g.wait()
        outs.append(pltpu.async_copy(buf, dst, osem))
    for o in outs:
        o.wait()

    @pl.when(is_mask_worker)
    def _():
        pltpu.make_async_copy(am_hbm, out_mask.at[:, pl.ds(0, T_TEXT)],
                              msem).wait()
        pltpu.make_async_copy(addm_hbm, out_mask.at[:, pl.ds(T_TEXT, T_ADD)],
                              msem).wait()


def kernel(input_ids, attention_mask, add_ids, add_mask, W):
    emb, mask = _gather_concat(input_ids, add_ids, attention_mask, add_mask,
                               W)
    return emb, mask
